# Initial kernel scaffold; baseline (speedup 1.0000x reference)
import jax
import jax.numpy as jnp
from jax.experimental import pallas as pl


def kernel(x_in, edge_index, edge_weight, lid_timeseries, W1, b1, Wlid, blid, W2, b2, W3, b3, W4, b4, g1, be1, g2, be2, g3, be3, g4, be4):
    raise NotImplementedError("write your pallas kernel here")


# speedup vs baseline: 7.0940x; 7.0940x over previous
import functools

import jax
import jax.numpy as jnp
from jax import lax
from jax.experimental import pallas as pl
from jax.experimental.pallas import tpu as pltpu
from jax.experimental.pallas import tpu_sc as plsc

_N = 10000
_E = 320000
_EPS = 1e-5

_NC, _NS, _NW = 2, 16, 32
_CH = 128
_NCHUNK = _E // _CH
_FULL = _NCHUNK // _NW
_REM = _NCHUNK - _FULL * _NW
_RPT = 632
_OUT_PIECES = [(0, 128), (128, 128), (256, 128), (384, 128), (512, 120)]


def _make_spmm(D):
    mesh = plsc.VectorSubcoreMesh(core_axis_name="c", subcore_axis_name="s")

    @functools.partial(
        pl.kernel,
        out_type=jax.ShapeDtypeStruct((_NC, _N, D), jnp.float32),
        mesh=mesh,
        scratch_types=[
            pltpu.VMEM((_CH,), jnp.int32),
            pltpu.VMEM((_CH,), jnp.int32),
            pltpu.VMEM((_CH,), jnp.float32),
            pltpu.VMEM((_CH, D), jnp.float32),
            pltpu.VMEM_SHARED((_N, D), jnp.float32),
            pltpu.SemaphoreType.DMA,
        ],
    )
    def spmm(hs_hbm, src_hbm, dst_hbm, w_hbm, out_hbm,
             src_v, dst_v, w_v, rows_v, acc_sh, sem):
        c = lax.axis_index("c")
        s = lax.axis_index("s")
        wid = s * _NC + c

        zero = jnp.zeros((16,), jnp.float32)

        def _zrow(i, _):
            for j in range(D // 16):
                rows_v[i, pl.ds(j * 16, 16)] = zero
            return 0

        lax.fori_loop(0, _CH, _zrow, 0)
        start = pl.multiple_of(jnp.minimum(s * _RPT, _N - _RPT), 8)
        for k, ln in _OUT_PIECES:
            pltpu.sync_copy(rows_v.at[pl.ds(0, ln)],
                            acc_sh.at[pl.ds(start + k, ln)])
        plsc.subcore_barrier()

        nt = _FULL + jnp.where(wid < _REM, 1, 0)

        def body(t, _):
            base = (wid + t * _NW) * _CH
            pltpu.sync_copy(src_hbm.at[pl.ds(base, _CH)], src_v)
            pltpu.sync_copy(dst_hbm.at[pl.ds(base, _CH)], dst_v)
            pltpu.sync_copy(w_hbm.at[pl.ds(base, _CH)], w_v)
            pltpu.async_copy(hs_hbm.at[src_v], rows_v, sem).wait()

            def sgrp(g, _):
                off = pl.multiple_of(g * 16, 16)
                w16 = w_v[pl.ds(off, 16)]
                for e in range(16):
                    wi = w16[e]
                    i = off + e
                    for j in range(D // 16):
                        rows_v[i, pl.ds(j * 16, 16)] = (
                            rows_v[i, pl.ds(j * 16, 16)] * wi)
                return 0

            lax.fori_loop(0, _CH // 16, sgrp, 0)
            pltpu.sync_copy(rows_v, acc_sh.at[dst_v], add=True)
            return 0

        lax.fori_loop(0, nt, body, 0)
        plsc.subcore_barrier()

        for k, ln in _OUT_PIECES:
            pltpu.sync_copy(acc_sh.at[pl.ds(start + k, ln)],
                            rows_v.at[pl.ds(0, ln)])
            pltpu.sync_copy(rows_v.at[pl.ds(0, ln)],
                            out_hbm.at[c, pl.ds(start + k, ln)])

    return spmm


_spmm128 = _make_spmm(128)


def _bn(y, g, be):
    mu = jnp.mean(y, axis=0, keepdims=True)
    var = jnp.mean((y - mu) ** 2, axis=0, keepdims=True)
    return (y - mu) * lax.rsqrt(var + _EPS) * g + be


def _tc1_body(acc0_ref, xin_ref, w1_ref, lid_ref, wlid_ref, blid_ref,
              dis_ref, h1s_ref, mlid_ref, keep_ref):
    deg = acc0_ref[0, :, 0:1] + acc0_ref[1, :, 0:1] + 1.0
    dis = lax.rsqrt(deg)
    dis_ref[...] = dis
    xin = xin_ref[...]
    h1s_ref[...] = dis * jnp.dot(xin, w1_ref[...],
                                 preferred_element_type=jnp.float32)
    keep = (xin[:, 0:1] != 0.0).astype(jnp.float32)
    keep_ref[...] = keep
    hlid = jax.nn.relu(jnp.dot(lid_ref[...], wlid_ref[...],
                               preferred_element_type=jnp.float32) + blid_ref[...])
    mlid_ref[...] = hlid * keep


def _tc2_body(acc_ref, dis_ref, h1s_ref, mlid_ref, keep_ref, g_ref, be_ref,
              w2_ref, h2sa_ref, h2sb_ref):
    dis = dis_ref[...]
    y1 = dis * (acc_ref[0] + acc_ref[1] + h1s_ref[...])
    x1 = jax.nn.relu(_bn(y1, g_ref[...], be_ref[...])) * keep_ref[...] + mlid_ref[...]
    w2 = w2_ref[...]
    h2sa_ref[...] = dis * jnp.dot(x1, w2[:, :128],
                                  preferred_element_type=jnp.float32)
    h2sb_ref[...] = dis * jnp.dot(x1, w2[:, 128:],
                                  preferred_element_type=jnp.float32)


def _tc3_body(acca_ref, accb_ref, dis_ref, h2sa_ref, h2sb_ref, g2_ref,
              be2_ref, w3_ref, h3s_ref):
    dis = dis_ref[...]
    y2a = dis * (acca_ref[0] + acca_ref[1] + h2sa_ref[...])
    y2b = dis * (accb_ref[0] + accb_ref[1] + h2sb_ref[...])
    y2 = jnp.concatenate([y2a, y2b], axis=1)
    x2 = jax.nn.relu(_bn(y2, g2_ref[...], be2_ref[...]))
    h3s_ref[...] = dis * jnp.dot(x2, w3_ref[...],
                                 preferred_element_type=jnp.float32)


def _tc4_body(acc_ref, dis_ref, h3s_ref, g3_ref, be3_ref, w4_ref, h4sp_ref):
    dis = dis_ref[...]
    y3 = dis * (acc_ref[0] + acc_ref[1] + h3s_ref[...])
    x3 = jax.nn.relu(_bn(y3, g3_ref[...], be3_ref[...]))
    h4 = dis * jnp.dot(x3, w4_ref[...], preferred_element_type=jnp.float32)
    h4sp_ref[...] = jnp.concatenate(
        [h4, jnp.zeros((_N, 104), jnp.float32)], axis=1)


def _tc5_body(acc_ref, dis_ref, h4sp_ref, g4_ref, be4_ref, x4_ref, sum_ref):
    dis = dis_ref[...]
    y4 = (dis * (acc_ref[0] + acc_ref[1] + h4sp_ref[...]))[:, :24]
    x4 = jax.nn.relu(_bn(y4, g4_ref[...], be4_ref[...]))
    x4_ref[...] = x4
    sum_ref[...] = jnp.sum(x4, keepdims=True)


def _tc(body, out_shape, *args):
    return pl.pallas_call(body, out_shape=out_shape)(*args)


def kernel(x_in, edge_index, edge_weight, lid_timeseries, W1, b1, Wlid, blid,
           W2, b2, W3, b3, W4, b4, g1, be1, g2, be2, g3, be3, g4, be4):
    f32 = jnp.float32
    src = edge_index[0].astype(jnp.int32)
    dst = edge_index[1].astype(jnp.int32)
    w = edge_weight.astype(f32)

    ones_tab = jnp.ones((_N, 128), f32)

    acc0 = _spmm128(ones_tab, src, dst, w)

    dis, h1s, mlid, keep = _tc(
        _tc1_body,
        (jax.ShapeDtypeStruct((_N, 1), f32),
         jax.ShapeDtypeStruct((_N, 128), f32),
         jax.ShapeDtypeStruct((_N, 128), f32),
         jax.ShapeDtypeStruct((_N, 1), f32)),
        acc0, x_in, W1, lid_timeseries, Wlid, blid.reshape(1, 128))

    acc1 = _spmm128(h1s, src, dst, w)
    h2sa, h2sb = _tc(
        _tc2_body,
        (jax.ShapeDtypeStruct((_N, 128), f32),
         jax.ShapeDtypeStruct((_N, 128), f32)),
        acc1, dis, h1s, mlid, keep, g1.reshape(1, 128), be1.reshape(1, 128), W2)

    acc2a = _spmm128(h2sa, src, dst, w)
    acc2b = _spmm128(h2sb, src, dst, w)
    h3s = _tc(_tc3_body, jax.ShapeDtypeStruct((_N, 128), f32),
              acc2a, acc2b, dis, h2sa, h2sb,
              g2.reshape(1, 256), be2.reshape(1, 256), W3)

    acc3 = _spmm128(h3s, src, dst, w)
    h4sp = _tc(_tc4_body, jax.ShapeDtypeStruct((_N, 128), f32),
               acc3, dis, h3s, g3.reshape(1, 128), be3.reshape(1, 128), W4)

    acc4 = _spmm128(h4sp, src, dst, w)
    x4, ssum = _tc(_tc5_body,
                   (jax.ShapeDtypeStruct((_N, 24), f32),
                    jax.ShapeDtypeStruct((1, 1), f32)),
                   acc4, dis, h4sp, g4.reshape(1, 24), be4.reshape(1, 24))

    return (x4, ssum[0, 0])


# speedup vs baseline: 10.3525x; 1.4593x over previous
import functools

import jax
import jax.numpy as jnp
from jax import lax
from jax.experimental import pallas as pl
from jax.experimental.pallas import tpu as pltpu
from jax.experimental.pallas import tpu_sc as plsc

_N = 10000
_NPAD = 10240
_E = 320000
_EPS = 1e-5

_NC, _NS, _NW = 2, 16, 32
_CH = 128
_NCHUNK = _E // _CH
_FULL = _NCHUNK // _NW
_REM = _NCHUNK - _FULL * _NW
_PAIRS = _FULL // 2
_RPT = 632
_OUT_PIECES = [(0, 128), (128, 128), (256, 128), (384, 128), (512, 120)]

_mesh = plsc.VectorSubcoreMesh(core_axis_name="c", subcore_axis_name="s")


def _scale_rows(rows_v, wf_v, D):
    def sgrp(g, _):
        off = pl.multiple_of(g * 16, 16)
        w16 = wf_v[0, pl.ds(off, 16)]
        for e in range(16):
            wi = w16[e]
            i = off + e
            for j in range(D // 16):
                rows_v[i, pl.ds(j * 16, 16)] = rows_v[i, pl.ds(j * 16, 16)] * wi
        return 0

    lax.fori_loop(0, _CH // 16, sgrp, 0)


def _make_spmm(D):
    @functools.partial(
        pl.kernel,
        out_type=jax.ShapeDtypeStruct((_NC, _N, D), jnp.float32),
        mesh=_mesh,
        scratch_types=[
            pltpu.VMEM((2, _CH), jnp.int32),
            pltpu.VMEM((2, _CH), jnp.int32),
            pltpu.VMEM((1, _CH), jnp.float32),
            pltpu.VMEM((1, _CH), jnp.float32),
            pltpu.VMEM((_CH, D), jnp.float32),
            pltpu.VMEM((_CH, D), jnp.float32),
            pltpu.VMEM_SHARED((_N, D), jnp.float32),
            pltpu.SemaphoreType.DMA,
            pltpu.SemaphoreType.DMA,
        ],
    )
    def spmm(hs_hbm, e2_hbm, wf_hbm, out_hbm,
             e2_v0, e2_v1, wf_v0, wf_v1, rows_v0, rows_v1, acc_sh,
             gs0, gs1):
        c = lax.axis_index("c")
        s = lax.axis_index("s")
        wid = s * _NC + c
        e2_v = (e2_v0, e2_v1)
        wf_v = (wf_v0, wf_v1)
        rows_v = (rows_v0, rows_v1)
        gsem = (gs0, gs1)

        zero = jnp.zeros((16,), jnp.float32)

        def _zrow(i, _):
            for j in range(D // 16):
                rows_v0[i, pl.ds(j * 16, 16)] = zero
            return 0

        lax.fori_loop(0, _CH, _zrow, 0)
        start = pl.multiple_of(jnp.minimum(s * _RPT, _N - _RPT), 8)
        for k, ln in _OUT_PIECES:
            pltpu.sync_copy(rows_v0.at[pl.ds(0, ln)],
                            acc_sh.at[pl.ds(start + k, ln)])
        plsc.subcore_barrier()

        def stage_and_gather(t, b):
            cid = wid + t * _NW
            pltpu.sync_copy(e2_hbm.at[cid], e2_v[b])
            pltpu.sync_copy(wf_hbm.at[cid], wf_v[b])
            return pltpu.async_copy(hs_hbm.at[e2_v[b].at[0]], rows_v[b],
                                    gsem[b])

        def finish(b, d):
            d.wait()
            _scale_rows(rows_v[b], wf_v[b], D)
            pltpu.sync_copy(rows_v[b], acc_sh.at[e2_v[b].at[1]], add=True)

        def pair(k, _):
            t0 = 2 * k
            d0 = stage_and_gather(t0, 0)
            d1 = stage_and_gather(t0 + 1, 1)
            finish(0, d0)
            finish(1, d1)
            return 0

        lax.fori_loop(0, _PAIRS, pair, 0)

        @pl.when(wid < _REM)
        def _():
            finish(0, stage_and_gather(_FULL, 0))

        plsc.subcore_barrier()

        for k, ln in _OUT_PIECES:
            pltpu.sync_copy(acc_sh.at[pl.ds(start + k, ln)],
                            rows_v0.at[pl.ds(0, ln)])
            pltpu.sync_copy(rows_v0.at[pl.ds(0, ln)],
                            out_hbm.at[c, pl.ds(start + k, ln)])

    return spmm


_spmm128 = _make_spmm(128)

def _bn(y, g, be):
    mu = jnp.mean(y, axis=0, keepdims=True)
    var = jnp.mean((y - mu) ** 2, axis=0, keepdims=True)
    return (y - mu) * lax.rsqrt(var + _EPS) * g + be


def _tc1_body(degp_ref, xin_ref, w1_ref, lid_ref, wlid_ref, blid_ref,
              dis_ref, h1s_ref, mlid_ref, keep_ref):
    deg = degp_ref[:, 0:1] + degp_ref[:, 1:2] + 1.0
    dis = lax.rsqrt(deg)
    dis_ref[...] = dis
    xin = xin_ref[...]
    h1s_ref[...] = dis * jnp.dot(xin, w1_ref[...],
                                 preferred_element_type=jnp.float32)
    keep = (xin[:, 0:1] != 0.0).astype(jnp.float32)
    keep_ref[...] = keep
    hlid = jax.nn.relu(jnp.dot(lid_ref[...], wlid_ref[...],
                               preferred_element_type=jnp.float32) + blid_ref[...])
    mlid_ref[...] = hlid * keep


def _tc2_body(acc_ref, dis_ref, h1s_ref, mlid_ref, keep_ref, g_ref, be_ref,
              w2_ref, h2sa_ref, h2sb_ref):
    dis = dis_ref[...]
    y1 = dis * (acc_ref[0] + acc_ref[1] + h1s_ref[...])
    x1 = jax.nn.relu(_bn(y1, g_ref[...], be_ref[...])) * keep_ref[...] + mlid_ref[...]
    w2 = w2_ref[...]
    h2sa_ref[...] = dis * jnp.dot(x1, w2[:, :128],
                                  preferred_element_type=jnp.float32)
    h2sb_ref[...] = dis * jnp.dot(x1, w2[:, 128:],
                                  preferred_element_type=jnp.float32)


def _tc3_body(acca_ref, accb_ref, dis_ref, h2sa_ref, h2sb_ref, g2_ref,
              be2_ref, w3_ref, h3s_ref):
    dis = dis_ref[...]
    y2a = dis * (acca_ref[0] + acca_ref[1] + h2sa_ref[...])
    y2b = dis * (accb_ref[0] + accb_ref[1] + h2sb_ref[...])
    y2 = jnp.concatenate([y2a, y2b], axis=1)
    x2 = jax.nn.relu(_bn(y2, g2_ref[...], be2_ref[...]))
    h3s_ref[...] = dis * jnp.dot(x2, w3_ref[...],
                                 preferred_element_type=jnp.float32)


def _tc4_body(acc_ref, dis_ref, h3s_ref, g3_ref, be3_ref, w4_ref, h4sp_ref):
    dis = dis_ref[...]
    y3 = dis * (acc_ref[0] + acc_ref[1] + h3s_ref[...])
    x3 = jax.nn.relu(_bn(y3, g3_ref[...], be3_ref[...]))
    h4 = dis * jnp.dot(x3, w4_ref[...], preferred_element_type=jnp.float32)
    h4sp_ref[...] = jnp.concatenate(
        [h4, jnp.zeros((_N, 104), jnp.float32)], axis=1)


def _tc5_body(acc_ref, dis_ref, h4sp_ref, g4_ref, be4_ref, x4_ref, sum_ref):
    dis = dis_ref[...]
    y4 = (dis * (acc_ref[0] + acc_ref[1] + h4sp_ref[...]))[:, :24]
    x4 = jax.nn.relu(_bn(y4, g4_ref[...], be4_ref[...]))
    x4_ref[...] = x4
    sum_ref[...] = jnp.sum(x4, keepdims=True)


def _tc(body, out_shape, *args):
    return pl.pallas_call(body, out_shape=out_shape)(*args)


def kernel(x_in, edge_index, edge_weight, lid_timeseries, W1, b1, Wlid, blid,
           W2, b2, W3, b3, W4, b4, g1, be1, g2, be2, g3, be3, g4, be4):
    f32 = jnp.float32
    src = edge_index[0].astype(jnp.int32).reshape(_NCHUNK, _CH)
    dst = edge_index[1].astype(jnp.int32).reshape(_NCHUNK, _CH)
    e2 = jnp.stack([src, dst], axis=1)
    wf = edge_weight.astype(f32).reshape(_NCHUNK, 1, _CH)

    ones_tab = jnp.ones((_N, 128), f32)
    degp = _spmm128(ones_tab, e2, wf)[:, :, 0]
    degp_t = degp.T

    dis, h1s, mlid, keep = _tc(
        _tc1_body,
        (jax.ShapeDtypeStruct((_N, 1), f32),
         jax.ShapeDtypeStruct((_N, 128), f32),
         jax.ShapeDtypeStruct((_N, 128), f32),
         jax.ShapeDtypeStruct((_N, 1), f32)),
        degp_t, x_in, W1, lid_timeseries, Wlid, blid.reshape(1, 128))

    acc1 = _spmm128(h1s, e2, wf)
    h2sa, h2sb = _tc(
        _tc2_body,
        (jax.ShapeDtypeStruct((_N, 128), f32),
         jax.ShapeDtypeStruct((_N, 128), f32)),
        acc1, dis, h1s, mlid, keep, g1.reshape(1, 128), be1.reshape(1, 128), W2)

    acc2a = _spmm128(h2sa, e2, wf)
    acc2b = _spmm128(h2sb, e2, wf)
    h3s = _tc(_tc3_body, jax.ShapeDtypeStruct((_N, 128), f32),
              acc2a, acc2b, dis, h2sa, h2sb,
              g2.reshape(1, 256), be2.reshape(1, 256), W3)

    acc3 = _spmm128(h3s, e2, wf)
    h4sp = _tc(_tc4_body, jax.ShapeDtypeStruct((_N, 128), f32),
               acc3, dis, h3s, g3.reshape(1, 128), be3.reshape(1, 128), W4)

    acc4 = _spmm128(h4sp, e2, wf)
    x4, ssum = _tc(_tc5_body,
                   (jax.ShapeDtypeStruct((_N, 24), f32),
                    jax.ShapeDtypeStruct((1, 1), f32)),
                   acc4, dis, h4sp, g4.reshape(1, 24), be4.reshape(1, 24))

    return (x4, ssum[0, 0])


# speedup vs baseline: 11.5217x; 1.1129x over previous
import functools

import jax
import jax.numpy as jnp
from jax import lax
from jax.experimental import pallas as pl
from jax.experimental.pallas import tpu as pltpu
from jax.experimental.pallas import tpu_sc as plsc

_N = 10000
_NPAD = 10240
_E = 320000
_EPS = 1e-5

_NC, _NS, _NW = 2, 16, 32
_CH = 128
_NCHUNK = _E // _CH
_FULL = _NCHUNK // _NW
_REM = _NCHUNK - _FULL * _NW
_PAIRS = _FULL // 2
_RPT = 632
_OUT_PIECES = [(0, 128), (128, 128), (256, 128), (384, 128), (512, 120)]

_mesh = plsc.VectorSubcoreMesh(core_axis_name="c", subcore_axis_name="s")


def _scale_rows(rows_v, wf_v, D):
    def sgrp(g, _):
        off = pl.multiple_of(g * 16, 16)
        w16 = wf_v[0, pl.ds(off, 16)]
        for e in range(16):
            wi = w16[e]
            i = off + e
            for j in range(D // 16):
                rows_v[i, pl.ds(j * 16, 16)] = rows_v[i, pl.ds(j * 16, 16)] * wi
        return 0

    lax.fori_loop(0, _CH // 16, sgrp, 0)


def _make_spmm(D, gather=True):
    nbuf = 3 if gather else 2

    @functools.partial(
        pl.kernel,
        out_type=jax.ShapeDtypeStruct((_NC, _N, D), jnp.float32),
        mesh=_mesh,
        scratch_types=(
            [pltpu.VMEM((2, _CH), jnp.int32) for _ in range(nbuf)]
            + [pltpu.VMEM((1, _CH), jnp.float32) for _ in range(nbuf)]
            + [pltpu.VMEM((_CH, D), jnp.float32) for _ in range(nbuf)]
            + [pltpu.VMEM_SHARED((_N, D), jnp.float32)]
            + [pltpu.SemaphoreType.DMA for _ in range(2 * nbuf)]
        ),
    )
    def spmm(hs_hbm, e2_hbm, wf_hbm, out_hbm, *refs):
        e2_v = refs[:nbuf]
        wf_v = refs[nbuf:2 * nbuf]
        rows_v = refs[2 * nbuf:3 * nbuf]
        acc_sh = refs[3 * nbuf]
        gsem = refs[3 * nbuf + 1:4 * nbuf + 1]
        isem = refs[4 * nbuf + 1:5 * nbuf + 1]
        c = lax.axis_index("c")
        s = lax.axis_index("s")
        wid = s * _NC + c

        zero = jnp.zeros((16,), jnp.float32)

        def _zbuf(r):
            def _zrow(i, _):
                for j in range(D // 16):
                    r[i, pl.ds(j * 16, 16)] = zero
                return 0
            lax.fori_loop(0, _CH, _zrow, 0)

        for b in range(nbuf):
            _zbuf(rows_v[b])
        start = pl.multiple_of(jnp.minimum(s * _RPT, _N - _RPT), 8)
        for k, ln in _OUT_PIECES:
            pltpu.sync_copy(rows_v[0].at[pl.ds(0, ln)],
                            acc_sh.at[pl.ds(start + k, ln)])
        plsc.subcore_barrier()

        def stage(t, b):
            cid = wid + t * _NW
            return (pltpu.async_copy(e2_hbm.at[cid], e2_v[b], isem[b]),
                    pltpu.async_copy(wf_hbm.at[cid], wf_v[b], isem[b]))

        def gather_rows(b):
            return pltpu.async_copy(hs_hbm.at[e2_v[b].at[0]], rows_v[b],
                                    gsem[b])

        def fill_rows(b):
            def _rep(g, _):
                off = pl.multiple_of(g * 16, 16)
                w16 = wf_v[b][0, pl.ds(off, 16)]
                for e in range(16):
                    rows_v[b][off + e, pl.ds(0, 16)] = w16 * 0.0 + w16[e]
                return 0
            lax.fori_loop(0, _CH // 16, _rep, 0)

        def scatter(b):
            pltpu.sync_copy(rows_v[b], acc_sh.at[e2_v[b].at[1]], add=True)

        if gather:
            def body(k, _):
                t0 = 3 * k
                ds = [stage(t0 + b, b) for b in range(3)]
                gs = []
                for b in range(3):
                    ds[b][0].wait()
                    ds[b][1].wait()
                    gs.append(gather_rows(b))
                for b in range(3):
                    gs[b].wait()
                    _scale_rows(rows_v[b], wf_v[b], D)
                    scatter(b)
                return 0

            lax.fori_loop(0, _FULL // 3, body, 0)

            @pl.when(wid < _REM)
            def _():
                d = stage(_FULL, 0)
                d[0].wait()
                d[1].wait()
                gather_rows(0).wait()
                _scale_rows(rows_v[0], wf_v[0], D)
                scatter(0)
        else:
            def body(k, _):
                t0 = 2 * k
                ds = [stage(t0 + b, b) for b in range(2)]
                for b in range(2):
                    ds[b][0].wait()
                    ds[b][1].wait()
                    fill_rows(b)
                    scatter(b)
                return 0

            lax.fori_loop(0, _FULL // 2, body, 0)

            @pl.when(wid < _REM)
            def _():
                d = stage(_FULL, 0)
                d[0].wait()
                d[1].wait()
                fill_rows(0)
                scatter(0)

        plsc.subcore_barrier()

        for k, ln in _OUT_PIECES:
            pltpu.sync_copy(acc_sh.at[pl.ds(start + k, ln)],
                            rows_v[0].at[pl.ds(0, ln)])
            pltpu.sync_copy(rows_v[0].at[pl.ds(0, ln)],
                            out_hbm.at[c, pl.ds(start + k, ln)])

    return spmm


_spmm128 = _make_spmm(128)
_degw = _make_spmm(128, gather=False)

def _bn(y, g, be):
    mu = jnp.mean(y, axis=0, keepdims=True)
    var = jnp.mean((y - mu) ** 2, axis=0, keepdims=True)
    return (y - mu) * lax.rsqrt(var + _EPS) * g + be


def _tc1_body(degp_ref, xin_ref, w1_ref, lid_ref, wlid_ref, blid_ref,
              dis_ref, h1s_ref, mlid_ref, keep_ref):
    deg = degp_ref[:, 0:1] + degp_ref[:, 1:2] + 1.0
    dis = lax.rsqrt(deg)
    dis_ref[...] = dis
    xin = xin_ref[...]
    h1s_ref[...] = dis * jnp.dot(xin, w1_ref[...],
                                 preferred_element_type=jnp.float32)
    keep = (xin[:, 0:1] != 0.0).astype(jnp.float32)
    keep_ref[...] = keep
    hlid = jax.nn.relu(jnp.dot(lid_ref[...], wlid_ref[...],
                               preferred_element_type=jnp.float32) + blid_ref[...])
    mlid_ref[...] = hlid * keep


def _tc2_body(acc_ref, dis_ref, h1s_ref, mlid_ref, keep_ref, g_ref, be_ref,
              w2_ref, h2sa_ref, h2sb_ref):
    dis = dis_ref[...]
    y1 = dis * (acc_ref[0] + acc_ref[1] + h1s_ref[...])
    x1 = jax.nn.relu(_bn(y1, g_ref[...], be_ref[...])) * keep_ref[...] + mlid_ref[...]
    w2 = w2_ref[...]
    h2sa_ref[...] = dis * jnp.dot(x1, w2[:, :128],
                                  preferred_element_type=jnp.float32)
    h2sb_ref[...] = dis * jnp.dot(x1, w2[:, 128:],
                                  preferred_element_type=jnp.float32)


def _tc3_body(acca_ref, accb_ref, dis_ref, h2sa_ref, h2sb_ref, g2_ref,
              be2_ref, w3_ref, h3s_ref):
    dis = dis_ref[...]
    y2a = dis * (acca_ref[0] + acca_ref[1] + h2sa_ref[...])
    y2b = dis * (accb_ref[0] + accb_ref[1] + h2sb_ref[...])
    y2 = jnp.concatenate([y2a, y2b], axis=1)
    x2 = jax.nn.relu(_bn(y2, g2_ref[...], be2_ref[...]))
    h3s_ref[...] = dis * jnp.dot(x2, w3_ref[...],
                                 preferred_element_type=jnp.float32)


def _tc4_body(acc_ref, dis_ref, h3s_ref, g3_ref, be3_ref, w4_ref, h4sp_ref):
    dis = dis_ref[...]
    y3 = dis * (acc_ref[0] + acc_ref[1] + h3s_ref[...])
    x3 = jax.nn.relu(_bn(y3, g3_ref[...], be3_ref[...]))
    h4 = dis * jnp.dot(x3, w4_ref[...], preferred_element_type=jnp.float32)
    h4sp_ref[...] = jnp.concatenate(
        [h4, jnp.zeros((_N, 104), jnp.float32)], axis=1)


def _tc5_body(acc_ref, dis_ref, h4sp_ref, g4_ref, be4_ref, x4_ref, sum_ref):
    dis = dis_ref[...]
    y4 = (dis * (acc_ref[0] + acc_ref[1] + h4sp_ref[...]))[:, :24]
    x4 = jax.nn.relu(_bn(y4, g4_ref[...], be4_ref[...]))
    x4_ref[...] = x4
    sum_ref[...] = jnp.sum(x4, keepdims=True)


def _tc(body, out_shape, *args):
    return pl.pallas_call(body, out_shape=out_shape)(*args)


def kernel(x_in, edge_index, edge_weight, lid_timeseries, W1, b1, Wlid, blid,
           W2, b2, W3, b3, W4, b4, g1, be1, g2, be2, g3, be3, g4, be4):
    f32 = jnp.float32
    src = edge_index[0].astype(jnp.int32).reshape(_NCHUNK, _CH)
    dst = edge_index[1].astype(jnp.int32).reshape(_NCHUNK, _CH)
    e2 = jnp.stack([src, dst], axis=1)
    wf = edge_weight.astype(f32).reshape(_NCHUNK, 1, _CH)

    dummy_tab = jnp.zeros((_N, 128), f32)
    degp = _degw(dummy_tab, e2, wf)[:, :, 0]
    degp_t = degp.T

    dis, h1s, mlid, keep = _tc(
        _tc1_body,
        (jax.ShapeDtypeStruct((_N, 1), f32),
         jax.ShapeDtypeStruct((_N, 128), f32),
         jax.ShapeDtypeStruct((_N, 128), f32),
         jax.ShapeDtypeStruct((_N, 1), f32)),
        degp_t, x_in, W1, lid_timeseries, Wlid, blid.reshape(1, 128))

    acc1 = _spmm128(h1s, e2, wf)
    h2sa, h2sb = _tc(
        _tc2_body,
        (jax.ShapeDtypeStruct((_N, 128), f32),
         jax.ShapeDtypeStruct((_N, 128), f32)),
        acc1, dis, h1s, mlid, keep, g1.reshape(1, 128), be1.reshape(1, 128), W2)

    acc2a = _spmm128(h2sa, e2, wf)
    acc2b = _spmm128(h2sb, e2, wf)
    h3s = _tc(_tc3_body, jax.ShapeDtypeStruct((_N, 128), f32),
              acc2a, acc2b, dis, h2sa, h2sb,
              g2.reshape(1, 256), be2.reshape(1, 256), W3)

    acc3 = _spmm128(h3s, e2, wf)
    h4sp = _tc(_tc4_body, jax.ShapeDtypeStruct((_N, 128), f32),
               acc3, dis, h3s, g3.reshape(1, 128), be3.reshape(1, 128), W4)

    acc4 = _spmm128(h4sp, e2, wf)
    x4, ssum = _tc(_tc5_body,
                   (jax.ShapeDtypeStruct((_N, 24), f32),
                    jax.ShapeDtypeStruct((1, 1), f32)),
                   acc4, dis, h4sp, g4.reshape(1, 24), be4.reshape(1, 24))

    return (x4, ssum[0, 0])


# speedup vs baseline: 12.7040x; 1.1026x over previous
import functools

import jax
import jax.numpy as jnp
from jax import lax
from jax.experimental import pallas as pl
from jax.experimental.pallas import tpu as pltpu
from jax.experimental.pallas import tpu_sc as plsc

_N = 10000
_NPAD = 10240
_E = 320000
_EPS = 1e-5

_NC, _NS, _NW = 2, 16, 32
_CH = 128
_NCHUNK = _E // _CH
_FULL = _NCHUNK // _NW
_REM = _NCHUNK - _FULL * _NW
_PAIRS = _FULL // 2
_RPT = 632
_OUT_PIECES = [(0, 128), (128, 128), (256, 128), (384, 128), (512, 120)]

_mesh = plsc.VectorSubcoreMesh(core_axis_name="c", subcore_axis_name="s")


def _scale_rows(rows_v, wf_v, D):
    def sgrp(g, _):
        off = pl.multiple_of(g * 16, 16)
        w16 = wf_v[0, pl.ds(off, 16)]
        for e in range(16):
            wi = w16[e]
            i = off + e
            for j in range(D // 16):
                rows_v[i, pl.ds(j * 16, 16)] = rows_v[i, pl.ds(j * 16, 16)] * wi
        return 0

    lax.fori_loop(0, _CH // 16, sgrp, 0)


def _make_spmm(D, gather=True):
    nbuf = 3 if gather else 2

    @functools.partial(
        pl.kernel,
        out_type=jax.ShapeDtypeStruct((_NC, _N, D), jnp.float32),
        mesh=_mesh,
        scratch_types=(
            [pltpu.VMEM((2, _CH), jnp.int32) for _ in range(nbuf)]
            + [pltpu.VMEM((1, _CH), jnp.float32) for _ in range(nbuf)]
            + [pltpu.VMEM((_CH, D), jnp.float32) for _ in range(nbuf)]
            + [pltpu.VMEM_SHARED((_N, D), jnp.float32)]
            + [pltpu.SemaphoreType.DMA for _ in range(3 * nbuf)]
        ),
    )
    def spmm(hs_hbm, e2_hbm, wf_hbm, out_hbm, *refs):
        e2_v = refs[:nbuf]
        wf_v = refs[nbuf:2 * nbuf]
        rows_v = refs[2 * nbuf:3 * nbuf]
        acc_sh = refs[3 * nbuf]
        gsem = refs[3 * nbuf + 1:4 * nbuf + 1]
        isem = refs[4 * nbuf + 1:5 * nbuf + 1]
        ssem = refs[5 * nbuf + 1:6 * nbuf + 1]
        c = lax.axis_index("c")
        s = lax.axis_index("s")
        wid = s * _NC + c

        zero = jnp.zeros((16,), jnp.float32)

        def _zbuf(r):
            def _zrow(i, _):
                for j in range(D // 16):
                    r[i, pl.ds(j * 16, 16)] = zero
                return 0
            lax.fori_loop(0, _CH, _zrow, 0)

        for b in range(nbuf):
            _zbuf(rows_v[b])
        start = pl.multiple_of(jnp.minimum(s * _RPT, _N - _RPT), 8)
        for k, ln in _OUT_PIECES:
            pltpu.sync_copy(rows_v[0].at[pl.ds(0, ln)],
                            acc_sh.at[pl.ds(start + k, ln)])
        plsc.subcore_barrier()

        def stage(t, b):
            cid = wid + t * _NW
            return (pltpu.async_copy(e2_hbm.at[cid], e2_v[b], isem[b]),
                    pltpu.async_copy(wf_hbm.at[cid], wf_v[b], isem[b]))

        def gather_rows(b):
            return pltpu.async_copy(hs_hbm.at[e2_v[b].at[0]], rows_v[b],
                                    gsem[b])

        def fill_rows(b):
            def _rep(g, _):
                off = pl.multiple_of(g * 16, 16)
                w16 = wf_v[b][0, pl.ds(off, 16)]
                for e in range(16):
                    rows_v[b][off + e, pl.ds(0, 16)] = w16 * 0.0 + w16[e]
                return 0
            lax.fori_loop(0, _CH // 16, _rep, 0)

        def scatter(b):
            return pltpu.async_copy(rows_v[b], acc_sh.at[e2_v[b].at[1]],
                                    ssem[b], add=True)

        if gather:
            def body(k, _):
                t0 = 3 * k
                ds = [stage(t0 + b, b) for b in range(3)]
                gs = []
                for b in range(3):
                    ds[b][0].wait()
                    ds[b][1].wait()
                    gs.append(gather_rows(b))
                sc = []
                for b in range(3):
                    gs[b].wait()
                    _scale_rows(rows_v[b], wf_v[b], D)
                    sc.append(scatter(b))
                for d in sc:
                    d.wait()
                return 0

            lax.fori_loop(0, _FULL // 3, body, 0)

            @pl.when(wid < _REM)
            def _():
                d = stage(_FULL, 0)
                d[0].wait()
                d[1].wait()
                gather_rows(0).wait()
                _scale_rows(rows_v[0], wf_v[0], D)
                scatter(0).wait()
        else:
            def body(k, _):
                t0 = 2 * k
                ds = [stage(t0 + b, b) for b in range(2)]
                sc = []
                for b in range(2):
                    ds[b][0].wait()
                    ds[b][1].wait()
                    fill_rows(b)
                    sc.append(scatter(b))
                for d in sc:
                    d.wait()
                return 0

            lax.fori_loop(0, _FULL // 2, body, 0)

            @pl.when(wid < _REM)
            def _():
                d = stage(_FULL, 0)
                d[0].wait()
                d[1].wait()
                fill_rows(0)
                scatter(0).wait()

        plsc.subcore_barrier()

        for k, ln in _OUT_PIECES:
            pltpu.sync_copy(acc_sh.at[pl.ds(start + k, ln)],
                            rows_v[0].at[pl.ds(0, ln)])
            pltpu.sync_copy(rows_v[0].at[pl.ds(0, ln)],
                            out_hbm.at[c, pl.ds(start + k, ln)])

    return spmm


_spmm128 = _make_spmm(128)
_degw = _make_spmm(128, gather=False)

def _bn(y, g, be):
    mu = jnp.mean(y, axis=0, keepdims=True)
    var = jnp.mean((y - mu) ** 2, axis=0, keepdims=True)
    return (y - mu) * lax.rsqrt(var + _EPS) * g + be


def _tc1_body(degp_ref, xin_ref, w1_ref, lid_ref, wlid_ref, blid_ref,
              dis_ref, h1s_ref, mlid_ref, keep_ref):
    deg = degp_ref[:, 0:1] + degp_ref[:, 1:2] + 1.0
    dis = lax.rsqrt(deg)
    dis_ref[...] = dis
    xin = xin_ref[...]
    h1s_ref[...] = dis * jnp.dot(xin, w1_ref[...],
                                 preferred_element_type=jnp.float32)
    keep = (xin[:, 0:1] != 0.0).astype(jnp.float32)
    keep_ref[...] = keep
    hlid = jax.nn.relu(jnp.dot(lid_ref[...], wlid_ref[...],
                               preferred_element_type=jnp.float32) + blid_ref[...])
    mlid_ref[...] = hlid * keep


def _tc2_body(acc_ref, dis_ref, h1s_ref, mlid_ref, keep_ref, g_ref, be_ref,
              w2_ref, h2sa_ref, h2sb_ref):
    dis = dis_ref[...]
    y1 = dis * (acc_ref[0] + acc_ref[1] + h1s_ref[...])
    x1 = jax.nn.relu(_bn(y1, g_ref[...], be_ref[...])) * keep_ref[...] + mlid_ref[...]
    w2 = w2_ref[...]
    h2sa_ref[...] = dis * jnp.dot(x1, w2[:, :128],
                                  preferred_element_type=jnp.float32)
    h2sb_ref[...] = dis * jnp.dot(x1, w2[:, 128:],
                                  preferred_element_type=jnp.float32)


def _tc3_body(acca_ref, accb_ref, dis_ref, h2sa_ref, h2sb_ref, g2_ref,
              be2_ref, w3_ref, h3s_ref):
    dis = dis_ref[...]
    y2a = dis * (acca_ref[0] + acca_ref[1] + h2sa_ref[...])
    y2b = dis * (accb_ref[0] + accb_ref[1] + h2sb_ref[...])
    y2 = jnp.concatenate([y2a, y2b], axis=1)
    x2 = jax.nn.relu(_bn(y2, g2_ref[...], be2_ref[...]))
    h3s_ref[...] = dis * jnp.dot(x2, w3_ref[...],
                                 preferred_element_type=jnp.float32)


def _tc4_body(acc_ref, dis_ref, h3s_ref, g3_ref, be3_ref, w4_ref, h4sp_ref):
    dis = dis_ref[...]
    y3 = dis * (acc_ref[0] + acc_ref[1] + h3s_ref[...])
    x3 = jax.nn.relu(_bn(y3, g3_ref[...], be3_ref[...]))
    h4 = dis * jnp.dot(x3, w4_ref[...], preferred_element_type=jnp.float32)
    h4sp_ref[...] = jnp.concatenate(
        [h4, jnp.zeros((_N, 104), jnp.float32)], axis=1)


def _tc5_body(acc_ref, dis_ref, h4sp_ref, g4_ref, be4_ref, x4_ref, sum_ref):
    dis = dis_ref[...]
    y4 = (dis * (acc_ref[0] + acc_ref[1] + h4sp_ref[...]))[:, :24]
    x4 = jax.nn.relu(_bn(y4, g4_ref[...], be4_ref[...]))
    x4_ref[...] = x4
    sum_ref[...] = jnp.sum(x4, keepdims=True)


def _tc(body, out_shape, *args):
    return pl.pallas_call(body, out_shape=out_shape)(*args)


def kernel(x_in, edge_index, edge_weight, lid_timeseries, W1, b1, Wlid, blid,
           W2, b2, W3, b3, W4, b4, g1, be1, g2, be2, g3, be3, g4, be4):
    f32 = jnp.float32
    src = edge_index[0].astype(jnp.int32).reshape(_NCHUNK, _CH)
    dst = edge_index[1].astype(jnp.int32).reshape(_NCHUNK, _CH)
    e2 = jnp.stack([src, dst], axis=1)
    wf = edge_weight.astype(f32).reshape(_NCHUNK, 1, _CH)

    dummy_tab = jnp.zeros((_N, 128), f32)
    degp = _degw(dummy_tab, e2, wf)[:, :, 0]
    degp_t = degp.T

    dis, h1s, mlid, keep = _tc(
        _tc1_body,
        (jax.ShapeDtypeStruct((_N, 1), f32),
         jax.ShapeDtypeStruct((_N, 128), f32),
         jax.ShapeDtypeStruct((_N, 128), f32),
         jax.ShapeDtypeStruct((_N, 1), f32)),
        degp_t, x_in, W1, lid_timeseries, Wlid, blid.reshape(1, 128))

    acc1 = _spmm128(h1s, e2, wf)
    h2sa, h2sb = _tc(
        _tc2_body,
        (jax.ShapeDtypeStruct((_N, 128), f32),
         jax.ShapeDtypeStruct((_N, 128), f32)),
        acc1, dis, h1s, mlid, keep, g1.reshape(1, 128), be1.reshape(1, 128), W2)

    acc2a = _spmm128(h2sa, e2, wf)
    acc2b = _spmm128(h2sb, e2, wf)
    h3s = _tc(_tc3_body, jax.ShapeDtypeStruct((_N, 128), f32),
              acc2a, acc2b, dis, h2sa, h2sb,
              g2.reshape(1, 256), be2.reshape(1, 256), W3)

    acc3 = _spmm128(h3s, e2, wf)
    h4sp = _tc(_tc4_body, jax.ShapeDtypeStruct((_N, 128), f32),
               acc3, dis, h3s, g3.reshape(1, 128), be3.reshape(1, 128), W4)

    acc4 = _spmm128(h4sp, e2, wf)
    x4, ssum = _tc(_tc5_body,
                   (jax.ShapeDtypeStruct((_N, 24), f32),
                    jax.ShapeDtypeStruct((1, 1), f32)),
                   acc4, dis, h4sp, g4.reshape(1, 24), be4.reshape(1, 24))

    return (x4, ssum[0, 0])
